# baseline (speedup 1.0000x reference)
import functools
import math

import jax
import jax.numpy as jnp
from jax import lax
from jax.experimental import pallas as pl
from jax.experimental.pallas import tpu as pltpu
from jax.experimental.pallas import tpu_sc as plsc

VOCAB = 32000
SMOOTH = 0.1
CONF = 1.0 - SMOOTH
SV = SMOOTH / (VOCAB - 2 + 1e-06)
LOG_SV = math.log(SV)
LOG_CONF = math.log(CONF)
ENT_BASE = (VOCAB - 2) * SV * LOG_SV
ENT_DELTA = CONF * LOG_CONF - SV * LOG_SV

ROWS = 4096
SC_ROWS = 2048
RB = 1024
VB = 3200

NC, NS, LANES = 2, 16, 16
NW = NC * NS
TILE_ROW_WORDS = 250 * 1024


CW = 6400
NQ = VOCAB // CW


def _sc_part_body(x_hbm, tgt_hbm, out_hbm, tgt_v, buf, res_v, *, g):
    wid = lax.axis_index("s") * NC + lax.axis_index("c")
    row0 = wid * g
    pltpu.sync_copy(tgt_hbm.at[pl.ds(row0, g)], tgt_v)

    iota = lax.broadcasted_iota(jnp.int32, (LANES,), 0)
    tgrps, trows, tqs, tcols, tmasks = [], [], [], [], []
    for v in range(g // LANES):
        t16 = tgt_v[pl.ds(v * LANES, LANES)]
        lr = v * LANES + iota
        tq = t16 // CW
        tgrps.append(jnp.right_shift(lr, 3))
        trows.append(jnp.bitwise_and(lr, 7))
        tqs.append(tq)
        tcols.append(t16 - tq * CW)
        tmasks.append(t16 >= 2)
    zeros16 = jnp.zeros((LANES,), jnp.int32)
    ones16 = zeros16 + 1

    def grp_body(tr8, carry):
        def q_body(q, carry):
            accs, xt_acc, edge_acc = carry
            pltpu.sync_copy(
                x_hbm.at[pl.ds(row0 + tr8 * 8, 8), pl.ds(q * CW, CW)], buf)

            for rr in range(8):
                def step(s, accs, rr=rr):
                    base = s * 256
                    return tuple(
                        accs[u] + buf[rr, pl.ds(base + u * LANES, LANES)]
                        for u in range(16)
                    )
                accs = lax.fori_loop(0, CW // 256, step, accs)

            for v in range(g // LANES):
                m = (tgrps[v] == tr8) & (tqs[v] == q) & tmasks[v]
                gval = plsc.load_gather(buf, [trows[v], tcols[v]], mask=m)
                xt_acc = xt_acc + jnp.where(m, gval, 0.0)
                me = (tgrps[v] == tr8) & (q == 0)
                ev0 = plsc.load_gather(buf, [trows[v], zeros16], mask=me)
                ev1 = plsc.load_gather(buf, [trows[v], ones16], mask=me)
                edge_acc = edge_acc + jnp.where(me, ev0 + ev1, 0.0)
            return accs, xt_acc, edge_acc

        return lax.fori_loop(0, NQ, q_body, carry)

    zero = jnp.zeros((LANES,), jnp.float32)
    accs, xt_acc, edge_acc = lax.fori_loop(
        0, g // 8, grp_body, ((zero,) * 16, zero, zero))
    total = functools.reduce(lambda a, b: a + b, accs)

    res_v[...] = total
    pltpu.sync_copy(res_v, out_hbm.at[pl.ds(wid * LANES, LANES)])
    res_v[...] = xt_acc
    pltpu.sync_copy(res_v, out_hbm.at[pl.ds(NW * LANES + wid * LANES, LANES)])
    res_v[...] = edge_acc
    pltpu.sync_copy(res_v, out_hbm.at[pl.ds(2 * NW * LANES + wid * LANES, LANES)])


def _sc_part(x2d, tgt1d):
    g = SC_ROWS // NW
    mesh = plsc.VectorSubcoreMesh(core_axis_name="c", subcore_axis_name="s")
    return pl.kernel(
        functools.partial(_sc_part_body, g=g),
        out_type=jax.ShapeDtypeStruct((3 * NW * LANES,), jnp.float32),
        mesh=mesh,
        scratch_types=[
            pltpu.VMEM((SC_ROWS // NW,), jnp.int32),
            pltpu.VMEM((8, CW), jnp.float32),
            pltpu.VMEM((LANES,), jnp.float32),
        ],
        compiler_params=pltpu.CompilerParams(needs_layout_passes=False),
    )(x2d, tgt1d)


def _tc_body(x_ref, tgt_ref, out_ref):
    i = pl.program_id(0)
    j = pl.program_id(1)
    nr = pl.num_programs(0)
    nv = pl.num_programs(1)

    @pl.when((i == 0) & (j == 0))
    def _init():
        out_ref[...] = jnp.zeros_like(out_ref)

    blk = x_ref[...]
    tgt = tgt_ref[0, pl.ds(SC_ROWS + i * RB, RB)]
    tloc = (tgt - j * VB)[:, None]
    scale = jnp.where(tgt[:, None] >= 2, CONF / SV, 1.0)
    cols = jax.lax.broadcasted_iota(jnp.int32, (RB, VB), 1)
    val = jnp.where(cols == tloc, blk * scale, blk)
    acc = -SV * jnp.sum(val)

    @pl.when(j == 0)
    def _edge():
        out_ref[...] = out_ref[...] + SV * jnp.sum(blk[:, 0] + blk[:, 1])

    @pl.when((i == nr - 1) & (j == nv - 1))
    def _entropy():
        t_all = tgt_ref[0, :]
        cnt = jnp.sum(jnp.where(t_all >= 2, 1.0, 0.0))
        out_ref[...] = out_ref[...] + (t_all.shape[0] * ENT_BASE + cnt * ENT_DELTA)

    out_ref[...] = out_ref[...] + acc


def _tc_call(x2d, tgt2d):
    skip = SC_ROWS // RB
    nr = (ROWS - SC_ROWS) // RB
    nv = VOCAB // VB
    return pl.pallas_call(
        _tc_body,
        grid=(nr, nv),
        in_specs=[
            pl.BlockSpec((RB, VB), lambda i, j: (i + skip, j)),
            pl.BlockSpec((1, ROWS), lambda i, j: (0, 0)),
        ],
        out_specs=pl.BlockSpec((1, 1), lambda i, j: (0, 0)),
        out_shape=jax.ShapeDtypeStruct((1, 1), jnp.float32),
    )(x2d, tgt2d)


def _combine_body(tc_ref, sc_ref, out_ref):
    n = NW * LANES
    s = jnp.sum(sc_ref[0, pl.ds(0, n)])
    xt = jnp.sum(sc_ref[0, pl.ds(n, n)])
    ed = jnp.sum(sc_ref[0, pl.ds(2 * n, n)])
    out_ref[...] = tc_ref[...] - SV * s - (CONF - SV) * xt + SV * ed


def _combine(tc_part, sc_parts):
    n = sc_parts.shape[0]
    out = pl.pallas_call(
        _combine_body,
        in_specs=[
            pl.BlockSpec((1, 1), lambda: (0, 0)),
            pl.BlockSpec((1, n), lambda: (0, 0)),
        ],
        out_specs=pl.BlockSpec((1, 1), lambda: (0, 0)),
        out_shape=jax.ShapeDtypeStruct((1, 1), jnp.float32),
    )(tc_part, sc_parts.reshape(1, n))
    return out[0, 0]


def kernel(x, target):
    x2d = x.reshape(ROWS, VOCAB)
    tgt1d = target.reshape(ROWS)
    sc_parts = _sc_part(x2d, tgt1d)
    tc_part = _tc_call(x2d, tgt1d.reshape(1, ROWS))
    return _combine(tc_part, sc_parts)


# speedup vs baseline: 1.2607x; 1.2607x over previous
import math

import jax
import jax.numpy as jnp
from jax.experimental import pallas as pl

VOCAB = 32000
SMOOTH = 0.1
CONF = 1.0 - SMOOTH
SV = SMOOTH / (VOCAB - 2 + 1e-06)
LOG_SV = math.log(SV)
LOG_CONF = math.log(CONF)
ENT_BASE = (VOCAB - 2) * SV * LOG_SV
ENT_DELTA = CONF * LOG_CONF - SV * LOG_SV

RB = 512
VB = 6400


def _loss_body(x_ref, tgt_ref, out_ref):
    i = pl.program_id(0)
    j = pl.program_id(1)
    nr = pl.num_programs(0)
    nv = pl.num_programs(1)

    @pl.when((i == 0) & (j == 0))
    def _init():
        out_ref[...] = jnp.zeros_like(out_ref)

    blk = x_ref[...]
    tgt = tgt_ref[0, pl.ds(i * RB, RB)]
    tloc = (tgt - j * VB)[:, None]
    scale = jnp.where(tgt[:, None] >= 2, CONF / SV, 1.0)
    cols = jax.lax.broadcasted_iota(jnp.int32, (RB, VB), 1)
    val = jnp.where(cols == tloc, blk * scale, blk)
    acc = -SV * jnp.sum(val)

    @pl.when(j == 0)
    def _edge():
        out_ref[...] = out_ref[...] + SV * jnp.sum(blk[:, 0] + blk[:, 1])

    @pl.when((i == nr - 1) & (j == nv - 1))
    def _entropy():
        t_all = tgt_ref[0, :]
        cnt = jnp.sum(jnp.where(t_all >= 2, 1.0, 0.0))
        out_ref[...] = out_ref[...] + (t_all.shape[0] * ENT_BASE + cnt * ENT_DELTA)

    out_ref[...] = out_ref[...] + acc


def kernel(x, target):
    rows = x.shape[0] * x.shape[1]
    x2d = x.reshape(rows, VOCAB)
    tgt2d = target.reshape(1, rows)
    nr = rows // RB
    nv = VOCAB // VB
    out = pl.pallas_call(
        _loss_body,
        grid=(nr, nv),
        in_specs=[
            pl.BlockSpec((RB, VB), lambda i, j: (i, j)),
            pl.BlockSpec((1, rows), lambda i, j: (0, 0)),
        ],
        out_specs=pl.BlockSpec((1, 1), lambda i, j: (0, 0)),
        out_shape=jax.ShapeDtypeStruct((1, 1), jnp.float32),
    )(x2d, tgt2d)
    return out[0, 0]


# speedup vs baseline: 1.2748x; 1.0112x over previous
import math

import jax
import jax.numpy as jnp
from jax.experimental import pallas as pl

VOCAB = 32000
SMOOTH = 0.1
CONF = 1.0 - SMOOTH
SV = SMOOTH / (VOCAB - 2 + 1e-06)
LOG_SV = math.log(SV)
LOG_CONF = math.log(CONF)
ENT_BASE = (VOCAB - 2) * SV * LOG_SV
ENT_DELTA = CONF * LOG_CONF - SV * LOG_SV

RB = 256
VB = 16000


def _loss_body(x_ref, tgt_ref, out_ref):
    i = pl.program_id(0)
    j = pl.program_id(1)
    nr = pl.num_programs(0)
    nv = pl.num_programs(1)

    @pl.when((i == 0) & (j == 0))
    def _init():
        out_ref[...] = jnp.zeros_like(out_ref)

    blk = x_ref[...]
    tgt = tgt_ref[0, pl.ds(i * RB, RB)]
    tloc = (tgt - j * VB)[:, None]
    scale = jnp.where(tgt[:, None] >= 2, CONF / SV, 1.0)
    cols = jax.lax.broadcasted_iota(jnp.int32, (RB, VB), 1)
    val = jnp.where(cols == tloc, blk * scale, blk)
    acc = -SV * jnp.sum(val)

    @pl.when(j == 0)
    def _edge():
        out_ref[...] = out_ref[...] + SV * jnp.sum(blk[:, 0] + blk[:, 1])

    @pl.when((i == nr - 1) & (j == nv - 1))
    def _entropy():
        t_all = tgt_ref[0, :]
        cnt = jnp.sum(jnp.where(t_all >= 2, 1.0, 0.0))
        out_ref[...] = out_ref[...] + (t_all.shape[0] * ENT_BASE + cnt * ENT_DELTA)

    out_ref[...] = out_ref[...] + acc


def kernel(x, target):
    rows = x.shape[0] * x.shape[1]
    x2d = x.reshape(rows, VOCAB)
    tgt2d = target.reshape(1, rows)
    nr = rows // RB
    nv = VOCAB // VB
    out = pl.pallas_call(
        _loss_body,
        grid=(nr, nv),
        in_specs=[
            pl.BlockSpec((RB, VB), lambda i, j: (i, j)),
            pl.BlockSpec((1, rows), lambda i, j: (0, 0)),
        ],
        out_specs=pl.BlockSpec((1, 1), lambda i, j: (0, 0)),
        out_shape=jax.ShapeDtypeStruct((1, 1), jnp.float32),
    )(x2d, tgt2d)
    return out[0, 0]


# speedup vs baseline: 1.3158x; 1.0322x over previous
import math

import jax
import jax.numpy as jnp
from jax.experimental import pallas as pl

VOCAB = 32000
SMOOTH = 0.1
CONF = 1.0 - SMOOTH
SV = SMOOTH / (VOCAB - 2 + 1e-06)
LOG_SV = math.log(SV)
LOG_CONF = math.log(CONF)
ENT_BASE = (VOCAB - 2) * SV * LOG_SV
ENT_DELTA = CONF * LOG_CONF - SV * LOG_SV

RB = 128
VB = 32000


def _loss_body(x_ref, tgt_ref, out_ref):
    i = pl.program_id(0)
    j = pl.program_id(1)
    nr = pl.num_programs(0)
    nv = pl.num_programs(1)

    @pl.when((i == 0) & (j == 0))
    def _init():
        out_ref[...] = jnp.zeros_like(out_ref)

    blk = x_ref[...]
    tgt = tgt_ref[0, pl.ds(i * RB, RB)]
    tloc = (tgt - j * VB)[:, None]
    scale = jnp.where(tgt[:, None] >= 2, CONF / SV, 1.0)
    cols = jax.lax.broadcasted_iota(jnp.int32, (RB, VB), 1)
    val = jnp.where(cols == tloc, blk * scale, blk)
    acc = -SV * jnp.sum(val)

    @pl.when(j == 0)
    def _edge():
        out_ref[...] = out_ref[...] + SV * jnp.sum(blk[:, 0] + blk[:, 1])

    @pl.when((i == nr - 1) & (j == nv - 1))
    def _entropy():
        t_all = tgt_ref[0, :]
        cnt = jnp.sum(jnp.where(t_all >= 2, 1.0, 0.0))
        out_ref[...] = out_ref[...] + (t_all.shape[0] * ENT_BASE + cnt * ENT_DELTA)

    out_ref[...] = out_ref[...] + acc


def kernel(x, target):
    rows = x.shape[0] * x.shape[1]
    x2d = x.reshape(rows, VOCAB)
    tgt2d = target.reshape(1, rows)
    nr = rows // RB
    nv = VOCAB // VB
    out = pl.pallas_call(
        _loss_body,
        grid=(nr, nv),
        in_specs=[
            pl.BlockSpec((RB, VB), lambda i, j: (i, j)),
            pl.BlockSpec((1, rows), lambda i, j: (0, 0)),
        ],
        out_specs=pl.BlockSpec((1, 1), lambda i, j: (0, 0)),
        out_shape=jax.ShapeDtypeStruct((1, 1), jnp.float32),
    )(x2d, tgt2d)
    return out[0, 0]
